# baseline (speedup 1.0000x reference)
import jax
import jax.numpy as jnp
from jax import lax
from jax.experimental import pallas as pl
from jax.experimental.pallas import tpu as pltpu
from jax.experimental.pallas import tpu_sc as plsc

_B = 2048
_S = 32
_BATCH = 4096
_C = 2 * _S

_NC = 2
_NS = 16
_NW = _NC * _NS
_RPW = _BATCH // _NW
_HALF = _RPW // 2

_TW = 128


def _repack(buf, packed):
    def _row(r, carry):
        for j in range(_C // 16):
            packed[r, pl.ds(j * 16, 16)] = buf[r, pl.ds(j * 16, 16)]
        return carry

    lax.fori_loop(0, _RPW, _row, 0)


def _sc_body(x_hbm, out_hbm, buf, packed, sem, semo):
    wid = lax.axis_index("s") * _NC + lax.axis_index("c")
    base = wid * _RPW
    pltpu.async_copy(
        x_hbm.at[pl.ds(base, _RPW), pl.ds(0, _TW)], buf, sem).wait()
    _repack(buf, packed)
    pltpu.async_copy(packed, out_hbm.at[pl.ds(base, _RPW)], semo).wait()


@jax.jit
def _paired_select(xB):
    mesh = plsc.VectorSubcoreMesh(
        core_axis_name="c", subcore_axis_name="s", num_cores=_NC
    )
    flat = pl.kernel(
        _sc_body,
        mesh=mesh,
        out_type=jax.ShapeDtypeStruct((_BATCH, _C), jnp.float32),
        scratch_types=[
            pltpu.VMEM((_RPW, _TW), jnp.float32),
            pltpu.VMEM((_RPW, _C), jnp.float32),
            pltpu.SemaphoreType.DMA,
            pltpu.SemaphoreType.DMA,
        ],
    )(xB)
    return flat.reshape(_BATCH, _S, 2)


def kernel(xB, PL, PR):
    return _paired_select(xB)


# speedup vs baseline: 1.0134x; 1.0134x over previous
import jax
import jax.numpy as jnp
from jax import lax
from jax.experimental import pallas as pl
from jax.experimental.pallas import tpu as pltpu
from jax.experimental.pallas import tpu_sc as plsc

_B = 2048
_S = 32
_BATCH = 4096
_C = 2 * _S

_NC = 2
_NS = 16
_NW = _NC * _NS
_RPW = _BATCH // _NW
_HALF = _RPW // 2

_TW = 128


def _repack(buf, packed):
    def _row(r, carry):
        for j in range(_C // 16):
            packed[r, pl.ds(j * 16, 16)] = buf[r, pl.ds(j * 16, 16)]
        return carry

    lax.fori_loop(0, _HALF, _row, 0)


def _sc_body(x_hbm, out_hbm, buf0, buf1, packed0, packed1, sem0, sem1, semo):
    wid = lax.axis_index("s") * _NC + lax.axis_index("c")
    base = wid * _RPW
    in0 = pltpu.async_copy(
        x_hbm.at[pl.ds(base, _HALF), pl.ds(0, _TW)], buf0, sem0)
    in1 = pltpu.async_copy(
        x_hbm.at[pl.ds(base + _HALF, _HALF), pl.ds(0, _TW)], buf1, sem1)
    in0.wait()
    _repack(buf0, packed0)
    out0 = pltpu.async_copy(packed0, out_hbm.at[pl.ds(base, _HALF)], semo)
    in1.wait()
    _repack(buf1, packed1)
    out1 = pltpu.async_copy(
        packed1, out_hbm.at[pl.ds(base + _HALF, _HALF)], semo)
    out0.wait()
    out1.wait()


@jax.jit
def _paired_select(xB):
    mesh = plsc.VectorSubcoreMesh(
        core_axis_name="c", subcore_axis_name="s", num_cores=_NC
    )
    flat = pl.kernel(
        _sc_body,
        mesh=mesh,
        out_type=jax.ShapeDtypeStruct((_BATCH, _C), jnp.float32),
        scratch_types=[
            pltpu.VMEM((_HALF, _TW), jnp.float32),
            pltpu.VMEM((_HALF, _TW), jnp.float32),
            pltpu.VMEM((_HALF, _C), jnp.float32),
            pltpu.VMEM((_HALF, _C), jnp.float32),
            pltpu.SemaphoreType.DMA,
            pltpu.SemaphoreType.DMA,
            pltpu.SemaphoreType.DMA,
        ],
    )(xB)
    return flat.reshape(_BATCH, _S, 2)


def kernel(xB, PL, PR):
    return _paired_select(xB)
